# baseline (speedup 1.0000x reference)
import functools

import jax
import jax.numpy as jnp
from jax import lax
from jax.experimental import pallas as pl
from jax.experimental.pallas import tpu as pltpu
from jax.experimental.pallas import tpu_sc as plsc

N_NODES = 10000
D = 128
DH = D // 2
NC = 2
NS = 16
ROWS_PER_TILE = 632
N_PAD = NS * ROWS_PER_TILE
PARK = 10008
CHUNK = 64
NB = 2
ZROWS = 79


def _sc_aggregate(fhalf, src2d, dst2d):
    n_rows = src2d.shape[0]
    cpt = n_rows // NS
    half = cpt // 2
    la = 1 if NB == 2 else 2
    n_groups = (cpt + la + NB - 1) // NB

    mesh = plsc.VectorSubcoreMesh(core_axis_name="c", subcore_axis_name="s")

    @functools.partial(
        pl.kernel,
        mesh=mesh,
        compiler_params=pltpu.CompilerParams(use_tc_tiling_on_sc=False),
        out_type=[
            jax.ShapeDtypeStruct((NC, N_PAD, DH), jnp.float32),
            jax.ShapeDtypeStruct((NC, N_PAD, 16), jnp.float32),
        ],
        scratch_types=(
            [
                pltpu.VMEM((cpt, CHUNK), jnp.int32),
                pltpu.VMEM((cpt, CHUNK), jnp.int32),
            ]
            + [pltpu.VMEM((CHUNK,), jnp.int32) for _ in range(NB)]
            + [pltpu.VMEM((CHUNK,), jnp.int32) for _ in range(NB)]
            + [pltpu.VMEM((CHUNK, DH), jnp.float32) for _ in range(NB)]
            + [
                pltpu.VMEM((CHUNK, 16), jnp.float32),
                pltpu.VMEM((ZROWS, DH), jnp.float32),
                pltpu.VMEM((ROWS_PER_TILE, 16), jnp.float32),
                pltpu.VMEM_SHARED((N_PAD, DH), jnp.float32),
                pltpu.VMEM_SHARED((N_PAD, 16), jnp.float32),
                pltpu.SemaphoreType.DMA,
            ]
            + [pltpu.SemaphoreType.DMA for _ in range(3 * NB)]
        ),
    )
    def agg(f2_hbm, src_hbm, dst_hbm, sums_hbm, deg_hbm, *refs):
        (src_all, dst_all) = refs[0:2]
        src_v = refs[2:2 + NB]
        dst_v = refs[2 + NB:2 + 2 * NB]
        rows = refs[2 + 2 * NB:2 + 3 * NB]
        ones_v, zbuf_v, zdeg_v, acc_sh, deg_sh, sem_i = refs[2 + 3 * NB:8 + 3 * NB]
        sem_g = refs[8 + 3 * NB:8 + 4 * NB]
        sem_s = refs[8 + 4 * NB:8 + 5 * NB]
        sem_d = refs[8 + 5 * NB:8 + 6 * NB]

        cid = lax.axis_index("c")
        sid = lax.axis_index("s")

        pltpu.async_copy(src_hbm.at[pl.ds(sid * cpt, cpt)], src_all, sem_i)
        pltpu.async_copy(dst_hbm.at[pl.ds(sid * cpt, cpt)], dst_all, sem_i)

        zeros16 = jnp.zeros((16,), jnp.float32)
        ones16 = jnp.ones((16,), jnp.float32)

        def zero_zbuf(i, _):
            for j in range(DH // 16):
                zbuf_v[i, pl.ds(j * 16, 16)] = zeros16
            return _
        lax.fori_loop(0, ZROWS, zero_zbuf, None)

        def zero_zdeg(i, _):
            zdeg_v[i, pl.ds(0, 16)] = zeros16
            return _
        lax.fori_loop(0, ROWS_PER_TILE, zero_zdeg, None)

        def fill_ones(i, _):
            ones_v[i, pl.ds(0, 16)] = ones16
            return _
        lax.fori_loop(0, CHUNK, fill_ones, None)

        for b in range(ROWS_PER_TILE // ZROWS):
            pltpu.sync_copy(zbuf_v,
                            acc_sh.at[pl.ds(sid * ROWS_PER_TILE + b * ZROWS, ZROWS)])
        pltpu.sync_copy(zdeg_v, deg_sh.at[pl.ds(sid * ROWS_PER_TILE, ROWS_PER_TILE)])

        pltpu.make_async_copy(src_hbm.at[pl.ds(sid * cpt, cpt)], src_all, sem_i).wait()
        pltpu.make_async_copy(dst_hbm.at[pl.ds(sid * cpt, cpt)], dst_all, sem_i).wait()
        plsc.subcore_barrier()

        def stage(c, k):
            for g in range(CHUNK // 16):
                src_v[k][pl.ds(g * 16, 16)] = (
                    src_all[c, pl.ds(g * 16, 16)] * 2 + cid)
                dst_v[k][pl.ds(g * 16, 16)] = dst_all[c, pl.ds(g * 16, 16)]

        def in_my_half(c):
            return (c < half) == (cid == 0)

        def issue_gather(b):
            pltpu.async_copy(f2_hbm.at[src_v[b]], rows[b], sem_g[b])

        for b in range(2):
            stage(b, b)
            issue_gather(b)

        def pair(g, _):
            for b in range(2):
                c = g * 2 + b

                @pl.when(c < cpt)
                def _():
                    pltpu.make_async_copy(f2_hbm.at[src_v[b]], rows[b],
                                          sem_g[b]).wait()
                    pltpu.sync_copy(rows[b], acc_sh.at[dst_v[b]], add=True)

                @pl.when(jnp.logical_and(c < cpt, in_my_half(c)))
                def _():
                    pltpu.sync_copy(ones_v, deg_sh.at[dst_v[b]], add=True)

                @pl.when(c + 2 < cpt)
                def _():
                    stage(c + 2, b)
                    issue_gather(b)
            return _
        lax.fori_loop(0, (cpt + 1) // 2, pair, None)

        plsc.subcore_barrier()
        pltpu.sync_copy(acc_sh.at[pl.ds(sid * ROWS_PER_TILE, ROWS_PER_TILE)],
                        sums_hbm.at[cid, pl.ds(sid * ROWS_PER_TILE, ROWS_PER_TILE)])
        pltpu.sync_copy(deg_sh.at[pl.ds(sid * ROWS_PER_TILE, ROWS_PER_TILE)],
                        deg_hbm.at[cid, pl.ds(sid * ROWS_PER_TILE, ROWS_PER_TILE)])

    return agg(fhalf, src2d, dst2d)


def _combine(feat, sums, deg, W_self, W_neigh, bias2d):
    R = 1000
    dn = (((1,), (1,)), ((), ()))

    def body(feat_ref, sums_ref, deg_ref, ws_ref, wn_ref, b_ref, o_ref):
        d = deg_ref[0][:, :1] + deg_ref[1][:, :1]
        s = jnp.concatenate([sums_ref[0], sums_ref[1]], axis=1)
        hn = s * (1.0 / jnp.maximum(d, 1.0))
        o_ref[...] = (
            lax.dot_general(feat_ref[...], ws_ref[...], dn,
                            preferred_element_type=jnp.float32)
            + lax.dot_general(hn, wn_ref[...], dn,
                              preferred_element_type=jnp.float32)
            + b_ref[...]
        )

    return pl.pallas_call(
        body,
        grid=(N_NODES // R,),
        in_specs=[
            pl.BlockSpec((R, D), lambda i: (i, 0)),
            pl.BlockSpec((NC, R, DH), lambda i: (0, i, 0)),
            pl.BlockSpec((NC, R, 16), lambda i: (0, i, 0)),
            pl.BlockSpec((D, D), lambda i: (0, 0)),
            pl.BlockSpec((D, D), lambda i: (0, 0)),
            pl.BlockSpec((1, D), lambda i: (0, 0)),
        ],
        out_specs=pl.BlockSpec((R, D), lambda i: (i, 0)),
        out_shape=jax.ShapeDtypeStruct((N_NODES, D), jnp.float32),
    )(feat, sums, deg, W_self, W_neigh, bias2d)


def kernel(feat, edge_index, W_self, W_neigh, bias):
    E = edge_index.shape[1]
    fhalf = feat.reshape(2 * N_NODES, DH)
    stride = NS * CHUNK
    e_pad = ((E + stride - 1) // stride) * stride
    pad = e_pad - E
    src = jnp.concatenate(
        [edge_index[0], jnp.zeros((pad,), jnp.int32)]) if pad else edge_index[0]
    dst = jnp.concatenate(
        [edge_index[1], jnp.full((pad,), PARK, jnp.int32)]) if pad else edge_index[1]
    src2d = src.reshape(e_pad // CHUNK, CHUNK)
    dst2d = dst.reshape(e_pad // CHUNK, CHUNK)
    sums, deg = _sc_aggregate(fhalf, src2d, dst2d)
    return _combine(feat, sums, deg, W_self, W_neigh, bias.reshape(1, D))


# speedup vs baseline: 1.1494x; 1.1494x over previous
import functools

import jax
import jax.numpy as jnp
from jax import lax
from jax.experimental import pallas as pl
from jax.experimental.pallas import tpu as pltpu
from jax.experimental.pallas import tpu_sc as plsc

N_NODES = 10000
D = 128
DH = D // 2
NC = 2
NS = 16
ROWS_PER_TILE = 632
N_PAD = NS * ROWS_PER_TILE
PARK = 10008
CHUNK = 80
NB = 2
ZROWS = 79


def _sc_aggregate(fhalf, src2d, dst2d):
    n_rows = src2d.shape[0]
    cpt = n_rows // NS
    half = cpt // 2
    la = 1 if NB == 2 else 2
    n_groups = (cpt + la + NB - 1) // NB

    mesh = plsc.VectorSubcoreMesh(core_axis_name="c", subcore_axis_name="s")

    @functools.partial(
        pl.kernel,
        mesh=mesh,
        compiler_params=pltpu.CompilerParams(use_tc_tiling_on_sc=False),
        out_type=[
            jax.ShapeDtypeStruct((NC, N_PAD, DH), jnp.float32),
            jax.ShapeDtypeStruct((NC, N_PAD, 16), jnp.float32),
        ],
        scratch_types=(
            [
                pltpu.VMEM((cpt, CHUNK), jnp.int32),
                pltpu.VMEM((cpt, CHUNK), jnp.int32),
            ]
            + [pltpu.VMEM((CHUNK,), jnp.int32) for _ in range(NB)]
            + [pltpu.VMEM((CHUNK,), jnp.int32) for _ in range(NB)]
            + [pltpu.VMEM((CHUNK, DH), jnp.float32) for _ in range(NB)]
            + [
                pltpu.VMEM((CHUNK, 16), jnp.float32),
                pltpu.VMEM((ZROWS, DH), jnp.float32),
                pltpu.VMEM((ROWS_PER_TILE, 16), jnp.float32),
                pltpu.VMEM_SHARED((N_PAD, DH), jnp.float32),
                pltpu.VMEM_SHARED((N_PAD, 16), jnp.float32),
                pltpu.SemaphoreType.DMA,
            ]
            + [pltpu.SemaphoreType.DMA for _ in range(3 * NB)]
        ),
    )
    def agg(f2_hbm, src_hbm, dst_hbm, sums_hbm, deg_hbm, *refs):
        (src_all, dst_all) = refs[0:2]
        src_v = refs[2:2 + NB]
        dst_v = refs[2 + NB:2 + 2 * NB]
        rows = refs[2 + 2 * NB:2 + 3 * NB]
        ones_v, zbuf_v, zdeg_v, acc_sh, deg_sh, sem_i = refs[2 + 3 * NB:8 + 3 * NB]
        sem_g = refs[8 + 3 * NB:8 + 4 * NB]
        sem_s = refs[8 + 4 * NB:8 + 5 * NB]
        sem_d = refs[8 + 5 * NB:8 + 6 * NB]

        cid = lax.axis_index("c")
        sid = lax.axis_index("s")

        pltpu.async_copy(src_hbm.at[pl.ds(sid * cpt, cpt)], src_all, sem_i)
        pltpu.async_copy(dst_hbm.at[pl.ds(sid * cpt, cpt)], dst_all, sem_i)

        zeros16 = jnp.zeros((16,), jnp.float32)
        ones16 = jnp.ones((16,), jnp.float32)

        def zero_zbuf(i, _):
            for j in range(DH // 16):
                zbuf_v[i, pl.ds(j * 16, 16)] = zeros16
            return _
        lax.fori_loop(0, ZROWS, zero_zbuf, None)

        def zero_zdeg(i, _):
            zdeg_v[i, pl.ds(0, 16)] = zeros16
            return _
        lax.fori_loop(0, ROWS_PER_TILE, zero_zdeg, None)

        def fill_ones(i, _):
            ones_v[i, pl.ds(0, 16)] = ones16
            return _
        lax.fori_loop(0, CHUNK, fill_ones, None)

        for b in range(ROWS_PER_TILE // ZROWS):
            pltpu.sync_copy(zbuf_v,
                            acc_sh.at[pl.ds(sid * ROWS_PER_TILE + b * ZROWS, ZROWS)])
        pltpu.sync_copy(zdeg_v, deg_sh.at[pl.ds(sid * ROWS_PER_TILE, ROWS_PER_TILE)])

        pltpu.make_async_copy(src_hbm.at[pl.ds(sid * cpt, cpt)], src_all, sem_i).wait()
        pltpu.make_async_copy(dst_hbm.at[pl.ds(sid * cpt, cpt)], dst_all, sem_i).wait()
        plsc.subcore_barrier()

        def stage(c, k):
            for g in range(CHUNK // 16):
                src_v[k][pl.ds(g * 16, 16)] = (
                    src_all[c, pl.ds(g * 16, 16)] * 2 + cid)
                dst_v[k][pl.ds(g * 16, 16)] = dst_all[c, pl.ds(g * 16, 16)]

        def in_my_half(c):
            return (c < half) == (cid == 0)

        def issue_gather(b):
            pltpu.async_copy(f2_hbm.at[src_v[b]], rows[b], sem_g[b])

        for b in range(2):
            stage(b, b)
            issue_gather(b)

        def pair(g, _):
            for b in range(2):
                c = g * 2 + b

                @pl.when(c < cpt)
                def _():
                    pltpu.make_async_copy(f2_hbm.at[src_v[b]], rows[b],
                                          sem_g[b]).wait()
                    pltpu.sync_copy(rows[b], acc_sh.at[dst_v[b]], add=True)

                @pl.when(jnp.logical_and(c < cpt, in_my_half(c)))
                def _():
                    pltpu.sync_copy(ones_v, deg_sh.at[dst_v[b]], add=True)

                @pl.when(c + 2 < cpt)
                def _():
                    stage(c + 2, b)
                    issue_gather(b)
            return _
        lax.fori_loop(0, (cpt + 1) // 2, pair, None)

        plsc.subcore_barrier()
        pltpu.sync_copy(acc_sh.at[pl.ds(sid * ROWS_PER_TILE, ROWS_PER_TILE)],
                        sums_hbm.at[cid, pl.ds(sid * ROWS_PER_TILE, ROWS_PER_TILE)])
        pltpu.sync_copy(deg_sh.at[pl.ds(sid * ROWS_PER_TILE, ROWS_PER_TILE)],
                        deg_hbm.at[cid, pl.ds(sid * ROWS_PER_TILE, ROWS_PER_TILE)])

    return agg(fhalf, src2d, dst2d)


def _combine(feat, sums, deg, W_self, W_neigh, bias2d):
    R = 1000
    dn = (((1,), (1,)), ((), ()))

    def body(feat_ref, sums_ref, deg_ref, ws_ref, wn_ref, b_ref, o_ref):
        d = deg_ref[0][:, :1] + deg_ref[1][:, :1]
        s = jnp.concatenate([sums_ref[0], sums_ref[1]], axis=1)
        hn = s * (1.0 / jnp.maximum(d, 1.0))
        o_ref[...] = (
            lax.dot_general(feat_ref[...], ws_ref[...], dn,
                            preferred_element_type=jnp.float32)
            + lax.dot_general(hn, wn_ref[...], dn,
                              preferred_element_type=jnp.float32)
            + b_ref[...]
        )

    return pl.pallas_call(
        body,
        grid=(N_NODES // R,),
        in_specs=[
            pl.BlockSpec((R, D), lambda i: (i, 0)),
            pl.BlockSpec((NC, R, DH), lambda i: (0, i, 0)),
            pl.BlockSpec((NC, R, 16), lambda i: (0, i, 0)),
            pl.BlockSpec((D, D), lambda i: (0, 0)),
            pl.BlockSpec((D, D), lambda i: (0, 0)),
            pl.BlockSpec((1, D), lambda i: (0, 0)),
        ],
        out_specs=pl.BlockSpec((R, D), lambda i: (i, 0)),
        out_shape=jax.ShapeDtypeStruct((N_NODES, D), jnp.float32),
    )(feat, sums, deg, W_self, W_neigh, bias2d)


def kernel(feat, edge_index, W_self, W_neigh, bias):
    E = edge_index.shape[1]
    fhalf = feat.reshape(2 * N_NODES, DH)
    stride = NS * CHUNK
    e_pad = ((E + stride - 1) // stride) * stride
    pad = e_pad - E
    src = jnp.concatenate(
        [edge_index[0], jnp.zeros((pad,), jnp.int32)]) if pad else edge_index[0]
    dst = jnp.concatenate(
        [edge_index[1], jnp.full((pad,), PARK, jnp.int32)]) if pad else edge_index[1]
    src2d = src.reshape(e_pad // CHUNK, CHUNK)
    dst2d = dst.reshape(e_pad // CHUNK, CHUNK)
    sums, deg = _sc_aggregate(fhalf, src2d, dst2d)
    return _combine(feat, sums, deg, W_self, W_neigh, bias.reshape(1, D))


# speedup vs baseline: 1.1755x; 1.0227x over previous
import functools

import jax
import jax.numpy as jnp
from jax import lax
from jax.experimental import pallas as pl
from jax.experimental.pallas import tpu as pltpu
from jax.experimental.pallas import tpu_sc as plsc

N_NODES = 10000
D = 128
DH = D // 2
NC = 2
NS = 16
ROWS_PER_TILE = 632
N_PAD = NS * ROWS_PER_TILE
PARK = 10008
CHUNK = 80
NB = 2
ZROWS = 79


def _sc_aggregate(fhalf, src2d, dst2d):
    n_rows = src2d.shape[0]
    cpt = n_rows // NS
    half = cpt // 2
    la = 1 if NB == 2 else 2
    n_groups = (cpt + la + NB - 1) // NB

    mesh = plsc.VectorSubcoreMesh(core_axis_name="c", subcore_axis_name="s")

    @functools.partial(
        pl.kernel,
        mesh=mesh,
        compiler_params=pltpu.CompilerParams(use_tc_tiling_on_sc=False),
        out_type=[
            jax.ShapeDtypeStruct((NC, N_PAD, DH), jnp.float32),
            jax.ShapeDtypeStruct((NC, N_PAD, 16), jnp.float32),
        ],
        scratch_types=(
            [
                pltpu.VMEM((cpt, CHUNK), jnp.int32),
                pltpu.VMEM((cpt, CHUNK), jnp.int32),
            ]
            + [pltpu.VMEM((CHUNK,), jnp.int32) for _ in range(NB)]
            + [pltpu.VMEM((CHUNK,), jnp.int32) for _ in range(NB)]
            + [pltpu.VMEM((CHUNK,), jnp.int32) for _ in range(NB)]
            + [pltpu.VMEM((CHUNK, DH), jnp.float32) for _ in range(NB)]
            + [
                pltpu.VMEM((CHUNK, 16), jnp.float32),
                pltpu.VMEM((ZROWS, DH), jnp.float32),
                pltpu.VMEM((ROWS_PER_TILE, 16), jnp.float32),
                pltpu.VMEM_SHARED((N_PAD, DH), jnp.float32),
                pltpu.VMEM_SHARED((N_PAD, 16), jnp.float32),
                pltpu.SemaphoreType.DMA,
            ]
            + [pltpu.SemaphoreType.DMA for _ in range(3 * NB)]
        ),
    )
    def agg(f2_hbm, src_hbm, dst_hbm, sums_hbm, deg_hbm, *refs):
        (src_all, dst_all) = refs[0:2]
        src_v = refs[2:2 + NB]
        dst_v = refs[2 + NB:2 + 2 * NB]
        dstd_v = refs[2 + 2 * NB:2 + 3 * NB]
        rows = refs[2 + 3 * NB:2 + 4 * NB]
        ones_v, zbuf_v, zdeg_v, acc_sh, deg_sh, sem_i = refs[2 + 4 * NB:8 + 4 * NB]
        sem_g = refs[8 + 4 * NB:8 + 5 * NB]
        sem_s = refs[8 + 5 * NB:8 + 6 * NB]
        sem_d = refs[8 + 6 * NB:8 + 7 * NB]

        cid = lax.axis_index("c")
        sid = lax.axis_index("s")

        pltpu.async_copy(src_hbm.at[pl.ds(sid * cpt, cpt)], src_all, sem_i)
        pltpu.async_copy(dst_hbm.at[pl.ds(sid * cpt, cpt)], dst_all, sem_i)

        zeros16 = jnp.zeros((16,), jnp.float32)
        ones16 = jnp.ones((16,), jnp.float32)

        def zero_zbuf(i, _):
            for j in range(DH // 16):
                zbuf_v[i, pl.ds(j * 16, 16)] = zeros16
            return _
        lax.fori_loop(0, ZROWS, zero_zbuf, None)

        def zero_zdeg(i, _):
            zdeg_v[i, pl.ds(0, 16)] = zeros16
            return _
        lax.fori_loop(0, ROWS_PER_TILE, zero_zdeg, None)

        def fill_ones(i, _):
            ones_v[i, pl.ds(0, 16)] = ones16
            return _
        lax.fori_loop(0, CHUNK, fill_ones, None)

        for b in range(ROWS_PER_TILE // ZROWS):
            pltpu.sync_copy(zbuf_v,
                            acc_sh.at[pl.ds(sid * ROWS_PER_TILE + b * ZROWS, ZROWS)])
        pltpu.sync_copy(zdeg_v, deg_sh.at[pl.ds(sid * ROWS_PER_TILE, ROWS_PER_TILE)])

        pltpu.make_async_copy(src_hbm.at[pl.ds(sid * cpt, cpt)], src_all, sem_i).wait()
        pltpu.make_async_copy(dst_hbm.at[pl.ds(sid * cpt, cpt)], dst_all, sem_i).wait()
        plsc.subcore_barrier()

        def stage(c, k):
            for g in range(CHUNK // 16):
                src_v[k][pl.ds(g * 16, 16)] = (
                    src_all[c, pl.ds(g * 16, 16)] * 2 + cid)
                dst_v[k][pl.ds(g * 16, 16)] = dst_all[c, pl.ds(g * 16, 16)]

        def in_my_half(c):
            return (c < half) == (cid == 0)

        def issue_gather(b):
            pltpu.async_copy(f2_hbm.at[src_v[b]], rows[b], sem_g[b])

        for b in range(2):
            stage(b, b)
            issue_gather(b)

        def pair(g, _):
            for b in range(2):
                c = g * 2 + b

                @pl.when(c < cpt)
                def _():
                    pltpu.make_async_copy(f2_hbm.at[src_v[b]], rows[b],
                                          sem_g[b]).wait()
                    pltpu.sync_copy(rows[b], acc_sh.at[dst_v[b]], add=True)

                @pl.when(jnp.logical_and(c < cpt, in_my_half(c)))
                def _():
                    @pl.when(jnp.logical_and(c - 2 >= 0, in_my_half(c - 2)))
                    def _():
                        pltpu.make_async_copy(ones_v, deg_sh.at[dstd_v[b]],
                                              sem_d[b]).wait()
                    for g in range(CHUNK // 16):
                        dstd_v[b][pl.ds(g * 16, 16)] = dst_v[b][pl.ds(g * 16, 16)]
                    pltpu.async_copy(ones_v, deg_sh.at[dstd_v[b]], sem_d[b],
                                     add=True)

                @pl.when(c + 2 < cpt)
                def _():
                    stage(c + 2, b)
                    issue_gather(b)
            return _
        lax.fori_loop(0, (cpt + 1) // 2, pair, None)

        for b in range(2):
            pltpu.make_async_copy(ones_v, deg_sh.at[dstd_v[b]], sem_d[b]).wait()

        plsc.subcore_barrier()
        pltpu.sync_copy(acc_sh.at[pl.ds(sid * ROWS_PER_TILE, ROWS_PER_TILE)],
                        sums_hbm.at[cid, pl.ds(sid * ROWS_PER_TILE, ROWS_PER_TILE)])
        pltpu.sync_copy(deg_sh.at[pl.ds(sid * ROWS_PER_TILE, ROWS_PER_TILE)],
                        deg_hbm.at[cid, pl.ds(sid * ROWS_PER_TILE, ROWS_PER_TILE)])

    return agg(fhalf, src2d, dst2d)


def _combine(feat, sums, deg, W_self, W_neigh, bias2d):
    R = 1000
    dn = (((1,), (1,)), ((), ()))

    def body(feat_ref, sums_ref, deg_ref, ws_ref, wn_ref, b_ref, o_ref):
        d = deg_ref[0][:, :1] + deg_ref[1][:, :1]
        s = jnp.concatenate([sums_ref[0], sums_ref[1]], axis=1)
        hn = s * (1.0 / jnp.maximum(d, 1.0))
        o_ref[...] = (
            lax.dot_general(feat_ref[...], ws_ref[...], dn,
                            preferred_element_type=jnp.float32)
            + lax.dot_general(hn, wn_ref[...], dn,
                              preferred_element_type=jnp.float32)
            + b_ref[...]
        )

    return pl.pallas_call(
        body,
        grid=(N_NODES // R,),
        in_specs=[
            pl.BlockSpec((R, D), lambda i: (i, 0)),
            pl.BlockSpec((NC, R, DH), lambda i: (0, i, 0)),
            pl.BlockSpec((NC, R, 16), lambda i: (0, i, 0)),
            pl.BlockSpec((D, D), lambda i: (0, 0)),
            pl.BlockSpec((D, D), lambda i: (0, 0)),
            pl.BlockSpec((1, D), lambda i: (0, 0)),
        ],
        out_specs=pl.BlockSpec((R, D), lambda i: (i, 0)),
        out_shape=jax.ShapeDtypeStruct((N_NODES, D), jnp.float32),
    )(feat, sums, deg, W_self, W_neigh, bias2d)


def kernel(feat, edge_index, W_self, W_neigh, bias):
    E = edge_index.shape[1]
    fhalf = feat.reshape(2 * N_NODES, DH)
    stride = NS * CHUNK
    e_pad = ((E + stride - 1) // stride) * stride
    pad = e_pad - E
    src = jnp.concatenate(
        [edge_index[0], jnp.zeros((pad,), jnp.int32)]) if pad else edge_index[0]
    dst = jnp.concatenate(
        [edge_index[1], jnp.full((pad,), PARK, jnp.int32)]) if pad else edge_index[1]
    src2d = src.reshape(e_pad // CHUNK, CHUNK)
    dst2d = dst.reshape(e_pad // CHUNK, CHUNK)
    sums, deg = _sc_aggregate(fhalf, src2d, dst2d)
    return _combine(feat, sums, deg, W_self, W_neigh, bias.reshape(1, D))


# speedup vs baseline: 1.4637x; 1.2452x over previous
import functools

import jax
import jax.numpy as jnp
from jax import lax
from jax.experimental import pallas as pl
from jax.experimental.pallas import tpu as pltpu
from jax.experimental.pallas import tpu_sc as plsc

N_NODES = 10000
D = 128
DH = D // 2
NC = 2
NS = 16
ROWS_PER_TILE = 632
N_PAD = NS * ROWS_PER_TILE
PARK = 10008
CHUNK = 80
NB = 3
ZROWS = 79


def _sc_aggregate(fhalf, src2d, dst2d):
    n_rows = src2d.shape[0]
    cpt = n_rows // NS
    half = cpt // 2
    la = 1 if NB == 2 else 2
    n_groups = (cpt + la + NB - 1) // NB

    mesh = plsc.VectorSubcoreMesh(core_axis_name="c", subcore_axis_name="s")

    @functools.partial(
        pl.kernel,
        mesh=mesh,
        compiler_params=pltpu.CompilerParams(use_tc_tiling_on_sc=False),
        out_type=[
            jax.ShapeDtypeStruct((NC, N_PAD, DH), jnp.float32),
            jax.ShapeDtypeStruct((NC, N_PAD, 16), jnp.float32),
        ],
        scratch_types=(
            [
                pltpu.VMEM((cpt, CHUNK), jnp.int32),
                pltpu.VMEM((cpt, CHUNK), jnp.int32),
            ]
            + [pltpu.VMEM((CHUNK,), jnp.int32) for _ in range(NB)]
            + [pltpu.VMEM((CHUNK,), jnp.int32) for _ in range(NB)]
            + [pltpu.VMEM((CHUNK,), jnp.int32) for _ in range(NB)]
            + [pltpu.VMEM((CHUNK, DH), jnp.float32) for _ in range(NB)]
            + [
                pltpu.VMEM((CHUNK, 16), jnp.float32),
                pltpu.VMEM((ZROWS, DH), jnp.float32),
                pltpu.VMEM((ROWS_PER_TILE, 16), jnp.float32),
                pltpu.VMEM_SHARED((N_PAD, DH), jnp.float32),
                pltpu.VMEM_SHARED((N_PAD, 16), jnp.float32),
                pltpu.SemaphoreType.DMA,
            ]
            + [pltpu.SemaphoreType.DMA for _ in range(3 * NB)]
        ),
    )
    def agg(f2_hbm, src_hbm, dst_hbm, sums_hbm, deg_hbm, *refs):
        (src_all, dst_all) = refs[0:2]
        src_v = refs[2:2 + NB]
        dst_v = refs[2 + NB:2 + 2 * NB]
        dstd_v = refs[2 + 2 * NB:2 + 3 * NB]
        rows = refs[2 + 3 * NB:2 + 4 * NB]
        ones_v, zbuf_v, zdeg_v, acc_sh, deg_sh, sem_i = refs[2 + 4 * NB:8 + 4 * NB]
        sem_g = refs[8 + 4 * NB:8 + 5 * NB]
        sem_s = refs[8 + 5 * NB:8 + 6 * NB]
        sem_d = refs[8 + 6 * NB:8 + 7 * NB]

        cid = lax.axis_index("c")
        sid = lax.axis_index("s")

        pltpu.async_copy(src_hbm.at[pl.ds(sid * cpt, cpt)], src_all, sem_i)
        pltpu.async_copy(dst_hbm.at[pl.ds(sid * cpt, cpt)], dst_all, sem_i)

        zeros16 = jnp.zeros((16,), jnp.float32)
        ones16 = jnp.ones((16,), jnp.float32)

        def zero_zbuf(i, _):
            for j in range(DH // 16):
                zbuf_v[i, pl.ds(j * 16, 16)] = zeros16
            return _
        lax.fori_loop(0, ZROWS, zero_zbuf, None)

        def zero_zdeg(i, _):
            zdeg_v[i, pl.ds(0, 16)] = zeros16
            return _
        lax.fori_loop(0, ROWS_PER_TILE, zero_zdeg, None)

        def fill_ones(i, _):
            ones_v[i, pl.ds(0, 16)] = ones16
            return _
        lax.fori_loop(0, CHUNK, fill_ones, None)

        for b in range(ROWS_PER_TILE // ZROWS):
            pltpu.sync_copy(zbuf_v,
                            acc_sh.at[pl.ds(sid * ROWS_PER_TILE + b * ZROWS, ZROWS)])
        pltpu.sync_copy(zdeg_v, deg_sh.at[pl.ds(sid * ROWS_PER_TILE, ROWS_PER_TILE)])

        pltpu.make_async_copy(src_hbm.at[pl.ds(sid * cpt, cpt)], src_all, sem_i).wait()
        pltpu.make_async_copy(dst_hbm.at[pl.ds(sid * cpt, cpt)], dst_all, sem_i).wait()
        plsc.subcore_barrier()

        def stage(c, k):
            for g in range(CHUNK // 16):
                src_v[k][pl.ds(g * 16, 16)] = (
                    src_all[c, pl.ds(g * 16, 16)] * 2 + cid)
                dst_v[k][pl.ds(g * 16, 16)] = dst_all[c, pl.ds(g * 16, 16)]

        def in_my_half(c):
            return (c < half) == (cid == 0)

        def issue_gather(b):
            pltpu.async_copy(f2_hbm.at[src_v[b]], rows[b], sem_g[b])

        def group(g, _):
            for k in range(NB):
                c = g * NB + k

                @pl.when(jnp.logical_and(c >= NB, c < cpt))
                def _():
                    pltpu.make_async_copy(rows[k], acc_sh.at[dst_v[k]],
                                          sem_s[k]).wait()

                @pl.when(c < cpt)
                def _():
                    stage(c, k)
                    issue_gather(k)

                kb = (k - la) % NB
                cb = c - la

                @pl.when(jnp.logical_and(cb >= 0, cb < cpt))
                def _():
                    pltpu.make_async_copy(f2_hbm.at[src_v[kb]], rows[kb],
                                          sem_g[kb]).wait()
                    pltpu.async_copy(rows[kb], acc_sh.at[dst_v[kb]], sem_s[kb],
                                     add=True)

                @pl.when(jnp.logical_and(jnp.logical_and(cb >= 0, cb < cpt),
                                         in_my_half(cb)))
                def _():
                    @pl.when(jnp.logical_and(cb - NB >= 0, in_my_half(cb - NB)))
                    def _():
                        pltpu.make_async_copy(ones_v, deg_sh.at[dstd_v[kb]],
                                              sem_d[kb]).wait()
                    for gg in range(CHUNK // 16):
                        dstd_v[kb][pl.ds(gg * 16, 16)] = (
                            dst_v[kb][pl.ds(gg * 16, 16)])
                    pltpu.async_copy(ones_v, deg_sh.at[dstd_v[kb]], sem_d[kb],
                                     add=True)
            return _
        lax.fori_loop(0, n_groups, group, None)

        for j in range(NB):
            k = (cpt - NB + j) % NB
            pltpu.make_async_copy(rows[k], acc_sh.at[dst_v[k]], sem_s[k]).wait()
        for b in range(NB):
            pltpu.make_async_copy(ones_v, deg_sh.at[dstd_v[b]], sem_d[b]).wait()

        plsc.subcore_barrier()
        pltpu.sync_copy(acc_sh.at[pl.ds(sid * ROWS_PER_TILE, ROWS_PER_TILE)],
                        sums_hbm.at[cid, pl.ds(sid * ROWS_PER_TILE, ROWS_PER_TILE)])
        pltpu.sync_copy(deg_sh.at[pl.ds(sid * ROWS_PER_TILE, ROWS_PER_TILE)],
                        deg_hbm.at[cid, pl.ds(sid * ROWS_PER_TILE, ROWS_PER_TILE)])

    return agg(fhalf, src2d, dst2d)


def _combine(feat, sums, deg, W_self, W_neigh, bias2d):
    R = 1000
    dn = (((1,), (1,)), ((), ()))

    def body(feat_ref, sums_ref, deg_ref, ws_ref, wn_ref, b_ref, o_ref):
        d = deg_ref[0][:, :1] + deg_ref[1][:, :1]
        s = jnp.concatenate([sums_ref[0], sums_ref[1]], axis=1)
        hn = s * (1.0 / jnp.maximum(d, 1.0))
        o_ref[...] = (
            lax.dot_general(feat_ref[...], ws_ref[...], dn,
                            preferred_element_type=jnp.float32)
            + lax.dot_general(hn, wn_ref[...], dn,
                              preferred_element_type=jnp.float32)
            + b_ref[...]
        )

    return pl.pallas_call(
        body,
        grid=(N_NODES // R,),
        in_specs=[
            pl.BlockSpec((R, D), lambda i: (i, 0)),
            pl.BlockSpec((NC, R, DH), lambda i: (0, i, 0)),
            pl.BlockSpec((NC, R, 16), lambda i: (0, i, 0)),
            pl.BlockSpec((D, D), lambda i: (0, 0)),
            pl.BlockSpec((D, D), lambda i: (0, 0)),
            pl.BlockSpec((1, D), lambda i: (0, 0)),
        ],
        out_specs=pl.BlockSpec((R, D), lambda i: (i, 0)),
        out_shape=jax.ShapeDtypeStruct((N_NODES, D), jnp.float32),
    )(feat, sums, deg, W_self, W_neigh, bias2d)


def kernel(feat, edge_index, W_self, W_neigh, bias):
    E = edge_index.shape[1]
    fhalf = feat.reshape(2 * N_NODES, DH)
    stride = NS * CHUNK
    e_pad = ((E + stride - 1) // stride) * stride
    pad = e_pad - E
    src = jnp.concatenate(
        [edge_index[0], jnp.zeros((pad,), jnp.int32)]) if pad else edge_index[0]
    dst = jnp.concatenate(
        [edge_index[1], jnp.full((pad,), PARK, jnp.int32)]) if pad else edge_index[1]
    src2d = src.reshape(e_pad // CHUNK, CHUNK)
    dst2d = dst.reshape(e_pad // CHUNK, CHUNK)
    sums, deg = _sc_aggregate(fhalf, src2d, dst2d)
    return _combine(feat, sums, deg, W_self, W_neigh, bias.reshape(1, D))


# speedup vs baseline: 1.5570x; 1.0637x over previous
import functools

import jax
import jax.numpy as jnp
from jax import lax
from jax.experimental import pallas as pl
from jax.experimental.pallas import tpu as pltpu
from jax.experimental.pallas import tpu_sc as plsc

N_NODES = 10000
D = 128
DH = D // 2
NC = 2
NS = 16
ROWS_PER_TILE = 632
N_PAD = NS * ROWS_PER_TILE
PARK = 10008
CHUNK = 80
NB = 4
ZROWS = 79


def _sc_aggregate(fhalf, src2d, dst2d):
    n_rows = src2d.shape[0]
    cpt = n_rows // NS
    half = cpt // 2
    la = 1 if NB == 2 else 2
    n_groups = (cpt + la + NB - 1) // NB

    mesh = plsc.VectorSubcoreMesh(core_axis_name="c", subcore_axis_name="s")

    @functools.partial(
        pl.kernel,
        mesh=mesh,
        compiler_params=pltpu.CompilerParams(use_tc_tiling_on_sc=False),
        out_type=[
            jax.ShapeDtypeStruct((NC, N_PAD, DH), jnp.float32),
            jax.ShapeDtypeStruct((NC, N_PAD, 16), jnp.float32),
        ],
        scratch_types=(
            [
                pltpu.VMEM((cpt, CHUNK), jnp.int32),
                pltpu.VMEM((cpt, CHUNK), jnp.int32),
            ]
            + [pltpu.VMEM((CHUNK,), jnp.int32) for _ in range(NB)]
            + [pltpu.VMEM((CHUNK,), jnp.int32) for _ in range(NB)]
            + [pltpu.VMEM((CHUNK,), jnp.int32) for _ in range(NB)]
            + [pltpu.VMEM((CHUNK, DH), jnp.float32) for _ in range(NB)]
            + [
                pltpu.VMEM((CHUNK, 16), jnp.float32),
                pltpu.VMEM((ZROWS, DH), jnp.float32),
                pltpu.VMEM((ROWS_PER_TILE, 16), jnp.float32),
                pltpu.VMEM_SHARED((N_PAD, DH), jnp.float32),
                pltpu.VMEM_SHARED((N_PAD, 16), jnp.float32),
                pltpu.SemaphoreType.DMA,
            ]
            + [pltpu.SemaphoreType.DMA for _ in range(3 * NB)]
        ),
    )
    def agg(f2_hbm, src_hbm, dst_hbm, sums_hbm, deg_hbm, *refs):
        (src_all, dst_all) = refs[0:2]
        src_v = refs[2:2 + NB]
        dst_v = refs[2 + NB:2 + 2 * NB]
        dstd_v = refs[2 + 2 * NB:2 + 3 * NB]
        rows = refs[2 + 3 * NB:2 + 4 * NB]
        ones_v, zbuf_v, zdeg_v, acc_sh, deg_sh, sem_i = refs[2 + 4 * NB:8 + 4 * NB]
        sem_g = refs[8 + 4 * NB:8 + 5 * NB]
        sem_s = refs[8 + 5 * NB:8 + 6 * NB]
        sem_d = refs[8 + 6 * NB:8 + 7 * NB]

        cid = lax.axis_index("c")
        sid = lax.axis_index("s")

        pltpu.async_copy(src_hbm.at[pl.ds(sid * cpt, cpt)], src_all, sem_i)
        pltpu.async_copy(dst_hbm.at[pl.ds(sid * cpt, cpt)], dst_all, sem_i)

        zeros16 = jnp.zeros((16,), jnp.float32)
        ones16 = jnp.ones((16,), jnp.float32)

        def zero_zbuf(i, _):
            for j in range(DH // 16):
                zbuf_v[i, pl.ds(j * 16, 16)] = zeros16
            return _
        lax.fori_loop(0, ZROWS, zero_zbuf, None)

        def zero_zdeg(i, _):
            zdeg_v[i, pl.ds(0, 16)] = zeros16
            return _
        lax.fori_loop(0, ROWS_PER_TILE, zero_zdeg, None)

        def fill_ones(i, _):
            ones_v[i, pl.ds(0, 16)] = ones16
            return _
        lax.fori_loop(0, CHUNK, fill_ones, None)

        for b in range(ROWS_PER_TILE // ZROWS):
            pltpu.sync_copy(zbuf_v,
                            acc_sh.at[pl.ds(sid * ROWS_PER_TILE + b * ZROWS, ZROWS)])
        pltpu.sync_copy(zdeg_v, deg_sh.at[pl.ds(sid * ROWS_PER_TILE, ROWS_PER_TILE)])

        pltpu.make_async_copy(src_hbm.at[pl.ds(sid * cpt, cpt)], src_all, sem_i).wait()
        pltpu.make_async_copy(dst_hbm.at[pl.ds(sid * cpt, cpt)], dst_all, sem_i).wait()
        plsc.subcore_barrier()

        def stage(c, k):
            for g in range(CHUNK // 16):
                src_v[k][pl.ds(g * 16, 16)] = (
                    src_all[c, pl.ds(g * 16, 16)] * 2 + cid)
                dst_v[k][pl.ds(g * 16, 16)] = dst_all[c, pl.ds(g * 16, 16)]

        def in_my_half(c):
            return (c < half) == (cid == 0)

        def issue_gather(b):
            pltpu.async_copy(f2_hbm.at[src_v[b]], rows[b], sem_g[b])

        def group(g, _):
            for k in range(NB):
                c = g * NB + k

                @pl.when(jnp.logical_and(c >= NB, c < cpt))
                def _():
                    pltpu.make_async_copy(rows[k], acc_sh.at[dst_v[k]],
                                          sem_s[k]).wait()

                @pl.when(c < cpt)
                def _():
                    stage(c, k)
                    issue_gather(k)

                kb = (k - la) % NB
                cb = c - la

                @pl.when(jnp.logical_and(cb >= 0, cb < cpt))
                def _():
                    pltpu.make_async_copy(f2_hbm.at[src_v[kb]], rows[kb],
                                          sem_g[kb]).wait()
                    pltpu.async_copy(rows[kb], acc_sh.at[dst_v[kb]], sem_s[kb],
                                     add=True)

                @pl.when(jnp.logical_and(jnp.logical_and(cb >= 0, cb < cpt),
                                         in_my_half(cb)))
                def _():
                    @pl.when(jnp.logical_and(cb - NB >= 0, in_my_half(cb - NB)))
                    def _():
                        pltpu.make_async_copy(ones_v, deg_sh.at[dstd_v[kb]],
                                              sem_d[kb]).wait()
                    for gg in range(CHUNK // 16):
                        dstd_v[kb][pl.ds(gg * 16, 16)] = (
                            dst_v[kb][pl.ds(gg * 16, 16)])
                    pltpu.async_copy(ones_v, deg_sh.at[dstd_v[kb]], sem_d[kb],
                                     add=True)
            return _
        lax.fori_loop(0, n_groups, group, None)

        for j in range(NB):
            k = (cpt - NB + j) % NB
            pltpu.make_async_copy(rows[k], acc_sh.at[dst_v[k]], sem_s[k]).wait()
        for b in range(NB):
            pltpu.make_async_copy(ones_v, deg_sh.at[dstd_v[b]], sem_d[b]).wait()

        plsc.subcore_barrier()
        pltpu.sync_copy(acc_sh.at[pl.ds(sid * ROWS_PER_TILE, ROWS_PER_TILE)],
                        sums_hbm.at[cid, pl.ds(sid * ROWS_PER_TILE, ROWS_PER_TILE)])
        pltpu.sync_copy(deg_sh.at[pl.ds(sid * ROWS_PER_TILE, ROWS_PER_TILE)],
                        deg_hbm.at[cid, pl.ds(sid * ROWS_PER_TILE, ROWS_PER_TILE)])

    return agg(fhalf, src2d, dst2d)


def _combine(feat, sums, deg, W_self, W_neigh, bias2d):
    R = 1000
    dn = (((1,), (1,)), ((), ()))

    def body(feat_ref, sums_ref, deg_ref, ws_ref, wn_ref, b_ref, o_ref):
        d = deg_ref[0][:, :1] + deg_ref[1][:, :1]
        s = jnp.concatenate([sums_ref[0], sums_ref[1]], axis=1)
        hn = s * (1.0 / jnp.maximum(d, 1.0))
        o_ref[...] = (
            lax.dot_general(feat_ref[...], ws_ref[...], dn,
                            preferred_element_type=jnp.float32)
            + lax.dot_general(hn, wn_ref[...], dn,
                              preferred_element_type=jnp.float32)
            + b_ref[...]
        )

    return pl.pallas_call(
        body,
        grid=(N_NODES // R,),
        in_specs=[
            pl.BlockSpec((R, D), lambda i: (i, 0)),
            pl.BlockSpec((NC, R, DH), lambda i: (0, i, 0)),
            pl.BlockSpec((NC, R, 16), lambda i: (0, i, 0)),
            pl.BlockSpec((D, D), lambda i: (0, 0)),
            pl.BlockSpec((D, D), lambda i: (0, 0)),
            pl.BlockSpec((1, D), lambda i: (0, 0)),
        ],
        out_specs=pl.BlockSpec((R, D), lambda i: (i, 0)),
        out_shape=jax.ShapeDtypeStruct((N_NODES, D), jnp.float32),
    )(feat, sums, deg, W_self, W_neigh, bias2d)


def kernel(feat, edge_index, W_self, W_neigh, bias):
    E = edge_index.shape[1]
    fhalf = feat.reshape(2 * N_NODES, DH)
    stride = NS * CHUNK
    e_pad = ((E + stride - 1) // stride) * stride
    pad = e_pad - E
    src = jnp.concatenate(
        [edge_index[0], jnp.zeros((pad,), jnp.int32)]) if pad else edge_index[0]
    dst = jnp.concatenate(
        [edge_index[1], jnp.full((pad,), PARK, jnp.int32)]) if pad else edge_index[1]
    src2d = src.reshape(e_pad // CHUNK, CHUNK)
    dst2d = dst.reshape(e_pad // CHUNK, CHUNK)
    sums, deg = _sc_aggregate(fhalf, src2d, dst2d)
    return _combine(feat, sums, deg, W_self, W_neigh, bias.reshape(1, D))


# speedup vs baseline: 1.5756x; 1.0119x over previous
import functools

import jax
import jax.numpy as jnp
from jax import lax
from jax.experimental import pallas as pl
from jax.experimental.pallas import tpu as pltpu
from jax.experimental.pallas import tpu_sc as plsc

N_NODES = 10000
D = 128
DH = D // 2
NC = 2
NS = 16
ROWS_PER_TILE = 632
N_PAD = NS * ROWS_PER_TILE
PARK = 10008
CHUNK = 80
NB = 4
ZROWS = 79


def _sc_aggregate(fhalf, src2d, dst2d):
    n_rows = src2d.shape[0]
    cpt = n_rows // NS
    half = cpt // 2
    la = 1 if NB == 2 else 2
    n_groups = (cpt + la + NB - 1) // NB

    mesh = plsc.VectorSubcoreMesh(core_axis_name="c", subcore_axis_name="s")

    @functools.partial(
        pl.kernel,
        mesh=mesh,
        compiler_params=pltpu.CompilerParams(use_tc_tiling_on_sc=False),
        out_type=[
            jax.ShapeDtypeStruct((NC, N_PAD, DH), jnp.float32),
            jax.ShapeDtypeStruct((NC, N_PAD, 16), jnp.float32),
        ],
        scratch_types=(
            [
                pltpu.VMEM((cpt, CHUNK), jnp.int32),
                pltpu.VMEM((cpt, CHUNK), jnp.int32),
            ]
            + [pltpu.VMEM((CHUNK,), jnp.int32) for _ in range(NB)]
            + [pltpu.VMEM((CHUNK,), jnp.int32) for _ in range(NB)]
            + [pltpu.VMEM((CHUNK,), jnp.int32) for _ in range(NB)]
            + [pltpu.VMEM((CHUNK, DH), jnp.float32) for _ in range(NB)]
            + [
                pltpu.VMEM((CHUNK, 16), jnp.float32),
                pltpu.VMEM((ZROWS, DH), jnp.float32),
                pltpu.VMEM((ROWS_PER_TILE, 16), jnp.float32),
                pltpu.VMEM_SHARED((N_PAD, DH), jnp.float32),
                pltpu.VMEM_SHARED((N_PAD, 16), jnp.float32),
                pltpu.SemaphoreType.DMA,
            ]
            + [pltpu.SemaphoreType.DMA for _ in range(3 * NB)]
        ),
    )
    def agg(f2_hbm, src_hbm, dst_hbm, sums_hbm, deg_hbm, *refs):
        (src_all, dst_all) = refs[0:2]
        src_v = refs[2:2 + NB]
        dst_v = refs[2 + NB:2 + 2 * NB]
        dstd_v = refs[2 + 2 * NB:2 + 3 * NB]
        rows = refs[2 + 3 * NB:2 + 4 * NB]
        ones_v, zbuf_v, zdeg_v, acc_sh, deg_sh, sem_i = refs[2 + 4 * NB:8 + 4 * NB]
        sem_g = refs[8 + 4 * NB:8 + 5 * NB]
        sem_s = refs[8 + 5 * NB:8 + 6 * NB]
        sem_d = refs[8 + 6 * NB:8 + 7 * NB]

        cid = lax.axis_index("c")
        sid = lax.axis_index("s")

        pltpu.async_copy(src_hbm.at[pl.ds(sid * cpt, cpt)], src_all, sem_i)
        pltpu.async_copy(dst_hbm.at[pl.ds(sid * cpt, cpt)], dst_all, sem_i)

        zeros16 = jnp.zeros((16,), jnp.float32)
        ones16 = jnp.ones((16,), jnp.float32)

        def zero_zbuf(i, _):
            for j in range(DH // 16):
                zbuf_v[i, pl.ds(j * 16, 16)] = zeros16
            return _
        lax.fori_loop(0, ZROWS, zero_zbuf, None)

        def zero_zdeg(i, _):
            zdeg_v[i, pl.ds(0, 16)] = zeros16
            return _
        lax.fori_loop(0, ROWS_PER_TILE, zero_zdeg, None)

        def fill_ones(i, _):
            ones_v[i, pl.ds(0, 16)] = ones16
            return _
        lax.fori_loop(0, CHUNK, fill_ones, None)

        for b in range(ROWS_PER_TILE // ZROWS):
            pltpu.sync_copy(zbuf_v,
                            acc_sh.at[pl.ds(sid * ROWS_PER_TILE + b * ZROWS, ZROWS)])
        pltpu.sync_copy(zdeg_v, deg_sh.at[pl.ds(sid * ROWS_PER_TILE, ROWS_PER_TILE)])

        pltpu.make_async_copy(src_hbm.at[pl.ds(sid * cpt, cpt)], src_all, sem_i).wait()
        pltpu.make_async_copy(dst_hbm.at[pl.ds(sid * cpt, cpt)], dst_all, sem_i).wait()
        plsc.subcore_barrier()

        def stage(c, k):
            for g in range(CHUNK // 16):
                src_v[k][pl.ds(g * 16, 16)] = (
                    src_all[c, pl.ds(g * 16, 16)] * 2 + cid)
                dst_v[k][pl.ds(g * 16, 16)] = dst_all[c, pl.ds(g * 16, 16)]

        def in_my_half(c):
            return (c < half) == (cid == 0)

        def issue_gather(b):
            pltpu.async_copy(f2_hbm.at[src_v[b]], rows[b], sem_g[b])

        def group(g, _):
            for k in range(NB):
                c = g * NB + k

                @pl.when(jnp.logical_and(c >= NB, c < cpt))
                def _():
                    pltpu.make_async_copy(rows[k], acc_sh.at[dst_v[k]],
                                          sem_s[k]).wait()

                @pl.when(c < cpt)
                def _():
                    stage(c, k)
                    issue_gather(k)

                kb = (k - la) % NB
                cb = c - la

                @pl.when(jnp.logical_and(cb >= 0, cb < cpt))
                def _():
                    pltpu.make_async_copy(f2_hbm.at[src_v[kb]], rows[kb],
                                          sem_g[kb]).wait()
                    pltpu.async_copy(rows[kb], acc_sh.at[dst_v[kb]], sem_s[kb],
                                     add=True)

                @pl.when(jnp.logical_and(jnp.logical_and(cb >= 0, cb < cpt),
                                         in_my_half(cb)))
                def _():
                    @pl.when(jnp.logical_and(cb - NB >= 0, in_my_half(cb - NB)))
                    def _():
                        pltpu.make_async_copy(ones_v, deg_sh.at[dstd_v[kb]],
                                              sem_d[kb]).wait()
                    for gg in range(CHUNK // 16):
                        dstd_v[kb][pl.ds(gg * 16, 16)] = (
                            dst_v[kb][pl.ds(gg * 16, 16)])
                    pltpu.async_copy(ones_v, deg_sh.at[dstd_v[kb]], sem_d[kb],
                                     add=True)
            return _
        lax.fori_loop(0, n_groups, group, None)

        for j in range(NB):
            k = (cpt - NB + j) % NB
            pltpu.make_async_copy(rows[k], acc_sh.at[dst_v[k]], sem_s[k]).wait()
        for b in range(NB):
            pltpu.make_async_copy(ones_v, deg_sh.at[dstd_v[b]], sem_d[b]).wait()

        plsc.subcore_barrier()
        pltpu.sync_copy(acc_sh.at[pl.ds(sid * ROWS_PER_TILE, ROWS_PER_TILE)],
                        sums_hbm.at[cid, pl.ds(sid * ROWS_PER_TILE, ROWS_PER_TILE)])
        pltpu.sync_copy(deg_sh.at[pl.ds(sid * ROWS_PER_TILE, ROWS_PER_TILE)],
                        deg_hbm.at[cid, pl.ds(sid * ROWS_PER_TILE, ROWS_PER_TILE)])

    return agg(fhalf, src2d, dst2d)


def _combine(feat, sums, deg, W_self, W_neigh, bias2d):
    R = 2000
    dn = (((1,), (1,)), ((), ()))

    def body(feat_ref, sums_ref, deg_ref, ws_ref, wn_ref, b_ref, o_ref):
        d = deg_ref[0][:, :1] + deg_ref[1][:, :1]
        s = jnp.concatenate([sums_ref[0], sums_ref[1]], axis=1)
        hn = s * (1.0 / jnp.maximum(d, 1.0))
        o_ref[...] = (
            lax.dot_general(feat_ref[...], ws_ref[...], dn,
                            preferred_element_type=jnp.float32)
            + lax.dot_general(hn, wn_ref[...], dn,
                              preferred_element_type=jnp.float32)
            + b_ref[...]
        )

    return pl.pallas_call(
        body,
        grid=(N_NODES // R,),
        in_specs=[
            pl.BlockSpec((R, D), lambda i: (i, 0)),
            pl.BlockSpec((NC, R, DH), lambda i: (0, i, 0)),
            pl.BlockSpec((NC, R, 16), lambda i: (0, i, 0)),
            pl.BlockSpec((D, D), lambda i: (0, 0)),
            pl.BlockSpec((D, D), lambda i: (0, 0)),
            pl.BlockSpec((1, D), lambda i: (0, 0)),
        ],
        out_specs=pl.BlockSpec((R, D), lambda i: (i, 0)),
        out_shape=jax.ShapeDtypeStruct((N_NODES, D), jnp.float32),
    )(feat, sums, deg, W_self, W_neigh, bias2d)


def kernel(feat, edge_index, W_self, W_neigh, bias):
    E = edge_index.shape[1]
    fhalf = feat.reshape(2 * N_NODES, DH)
    stride = NS * CHUNK
    e_pad = ((E + stride - 1) // stride) * stride
    pad = e_pad - E
    src = jnp.concatenate(
        [edge_index[0], jnp.zeros((pad,), jnp.int32)]) if pad else edge_index[0]
    dst = jnp.concatenate(
        [edge_index[1], jnp.full((pad,), PARK, jnp.int32)]) if pad else edge_index[1]
    src2d = src.reshape(e_pad // CHUNK, CHUNK)
    dst2d = dst.reshape(e_pad // CHUNK, CHUNK)
    sums, deg = _sc_aggregate(fhalf, src2d, dst2d)
    return _combine(feat, sums, deg, W_self, W_neigh, bias.reshape(1, D))
